# Initial kernel scaffold; baseline (speedup 1.0000x reference)
import functools

import jax
import jax.numpy as jnp
from jax import lax
from jax.experimental import pallas as pl
from jax.experimental.pallas import tpu as pltpu
from jax.experimental.pallas import tpu_sc as plsc

ALPHABET_SIZE = 500000
D_FEAT = 32

_info = plsc.get_sparse_core_info()
_NC, _NS, _L = _info.num_cores, _info.num_subcores, _info.num_lanes
_NW = _NC * _NS

_B = 4096 * 200
_B_PER_W = _B // _NW
_CHUNK = 1600
_N_CHUNKS = _B_PER_W // _CHUNK


@functools.partial(
    pl.kernel,
    mesh=plsc.VectorSubcoreMesh(core_axis_name="c", subcore_axis_name="s"),
    out_type=jax.ShapeDtypeStruct((_B, D_FEAT), jnp.float32),
    scratch_types=[
        pltpu.VMEM((_CHUNK,), jnp.int32),
        pltpu.VMEM((_CHUNK, D_FEAT), jnp.float32),
        pltpu.SemaphoreType.DMA,
    ],
)
def _embed_gather(idx_hbm, table_hbm, out_hbm, idx_v, rows_v, sem):
    wid = lax.axis_index("s") * _NC + lax.axis_index("c")
    base = wid * _B_PER_W

    def chunk_body(g, carry):
        off = base + g * _CHUNK
        pltpu.sync_copy(idx_hbm.at[pl.ds(off, _CHUNK)], idx_v)

        def add_body(i, c):
            sl = pl.ds(i * _L, _L)
            idx_v[sl] = idx_v[sl] + ALPHABET_SIZE
            return c

        lax.fori_loop(0, _CHUNK // _L, add_body, 0)
        pltpu.async_copy(table_hbm.at[idx_v], rows_v, sem).wait()
        pltpu.sync_copy(rows_v, out_hbm.at[pl.ds(off, _CHUNK)])
        return carry

    lax.fori_loop(0, _N_CHUNKS, chunk_body, 0)


def kernel(literal, embedding):
    flat = literal.reshape(-1)
    out = _embed_gather(flat, embedding)
    return out.reshape(literal.shape[0], literal.shape[1], D_FEAT)


# speedup vs baseline: 1.4679x; 1.4679x over previous
import functools

import jax
import jax.numpy as jnp
from jax import lax
from jax.experimental import pallas as pl
from jax.experimental.pallas import tpu as pltpu
from jax.experimental.pallas import tpu_sc as plsc

ALPHABET_SIZE = 500000
D_FEAT = 32

_info = plsc.get_sparse_core_info()
_NC, _NS, _L = _info.num_cores, _info.num_subcores, _info.num_lanes
_NW = _NC * _NS

_B = 4096 * 200
_B_PER_W = _B // _NW
_CHUNK = 1600
_N_CHUNKS = _B_PER_W // _CHUNK


@functools.partial(
    pl.kernel,
    mesh=plsc.VectorSubcoreMesh(core_axis_name="c", subcore_axis_name="s"),
    out_type=jax.ShapeDtypeStruct((_B, D_FEAT), jnp.float32),
    scratch_types=[
        pltpu.VMEM((_CHUNK,), jnp.int32),
        pltpu.VMEM((_CHUNK, D_FEAT), jnp.float32),
        pltpu.SemaphoreType.DMA,
    ],
    compiler_params=pltpu.CompilerParams(use_tc_tiling_on_sc=False),
)
def _embed_gather(idx_hbm, table_hbm, out_hbm, idx_v, rows_v, sem):
    wid = lax.axis_index("s") * _NC + lax.axis_index("c")
    base = wid * _B_PER_W

    def chunk_body(g, carry):
        off = base + g * _CHUNK
        pltpu.sync_copy(idx_hbm.at[pl.ds(off, _CHUNK)], idx_v)

        def add_body(i, c):
            sl = pl.ds(i * _L, _L)
            idx_v[sl] = idx_v[sl] + ALPHABET_SIZE
            return c

        lax.fori_loop(0, _CHUNK // _L, add_body, 0)
        pltpu.async_copy(table_hbm.at[idx_v], rows_v, sem).wait()
        pltpu.sync_copy(rows_v, out_hbm.at[pl.ds(off, _CHUNK)])
        return carry

    lax.fori_loop(0, _N_CHUNKS, chunk_body, 0)


def kernel(literal, embedding):
    flat = literal.reshape(-1)
    out = _embed_gather(flat, embedding)
    return out.reshape(literal.shape[0], literal.shape[1], D_FEAT)


# speedup vs baseline: 1.4793x; 1.0078x over previous
import functools

import jax
import jax.numpy as jnp
from jax import lax
from jax.experimental import pallas as pl
from jax.experimental.pallas import tpu as pltpu
from jax.experimental.pallas import tpu_sc as plsc

ALPHABET_SIZE = 500000
D_FEAT = 32

_info = plsc.get_sparse_core_info()
_NC, _NS, _L = _info.num_cores, _info.num_subcores, _info.num_lanes
_NW = _NC * _NS

_B = 4096 * 200
_B_PER_W = _B // _NW
_CHUNK = 1600
_N_CHUNKS = _B_PER_W // _CHUNK


@functools.partial(
    pl.kernel,
    mesh=plsc.VectorSubcoreMesh(core_axis_name="c", subcore_axis_name="s"),
    out_type=jax.ShapeDtypeStruct((_B, D_FEAT), jnp.float32),
    scratch_types=[
        pltpu.VMEM((_CHUNK,), jnp.int32),
        pltpu.VMEM((_CHUNK, D_FEAT), jnp.float32),
        pltpu.SemaphoreType.DMA,
    ],
    compiler_params=pltpu.CompilerParams(use_tc_tiling_on_sc=False),
)
def _embed_gather(idx_hbm, table_hbm, out_hbm, idx_v, rows_v, sem):
    wid = lax.axis_index("s") * _NC + lax.axis_index("c")
    base = wid * _B_PER_W

    def chunk_body(g, carry):
        off = base + g * _CHUNK
        pltpu.sync_copy(idx_hbm.at[pl.ds(off, _CHUNK)], idx_v)

        pltpu.async_copy(
            table_hbm.at[pl.ds(ALPHABET_SIZE, ALPHABET_SIZE)].at[idx_v],
            rows_v, sem).wait()
        pltpu.sync_copy(rows_v, out_hbm.at[pl.ds(off, _CHUNK)])
        return carry

    lax.fori_loop(0, _N_CHUNKS, chunk_body, 0)


def kernel(literal, embedding):
    flat = literal.reshape(-1)
    out = _embed_gather(flat, embedding)
    return out.reshape(literal.shape[0], literal.shape[1], D_FEAT)


# speedup vs baseline: 1.4941x; 1.0100x over previous
import functools

import jax
import jax.numpy as jnp
from jax import lax
from jax.experimental import pallas as pl
from jax.experimental.pallas import tpu as pltpu
from jax.experimental.pallas import tpu_sc as plsc

ALPHABET_SIZE = 500000
D_FEAT = 32

_info = plsc.get_sparse_core_info()
_NC, _NS, _L = _info.num_cores, _info.num_subcores, _info.num_lanes
_NW = _NC * _NS

_B = 4096 * 200
_B_PER_W = _B // _NW
_CHUNK = 1600
_N_CHUNKS = _B_PER_W // _CHUNK


@functools.partial(
    pl.kernel,
    mesh=plsc.VectorSubcoreMesh(core_axis_name="c", subcore_axis_name="s"),
    out_type=jax.ShapeDtypeStruct((_B, D_FEAT), jnp.float32),
    scratch_types=[
        pltpu.VMEM((2, _CHUNK), jnp.int32),
        pltpu.VMEM((2, _CHUNK, D_FEAT), jnp.float32),
        pltpu.SemaphoreType.DMA,
        pltpu.SemaphoreType.DMA,
        pltpu.SemaphoreType.DMA,
        pltpu.SemaphoreType.DMA,
    ],
    compiler_params=pltpu.CompilerParams(use_tc_tiling_on_sc=False),
)
def _embed_gather(idx_hbm, table_hbm, out_hbm, idx_v, rows_v, g0, g1, s0, s1):
    wid = lax.axis_index("s") * _NC + lax.axis_index("c")
    base = wid * _B_PER_W
    table = table_hbm.at[pl.ds(ALPHABET_SIZE, ALPHABET_SIZE)]
    gsem = [g0, g1]
    ssem = [s0, s1]

    gathers = [None, None]
    stores = [None, None]
    for g in range(_N_CHUNKS):
        b = g % 2
        off = base + g * _CHUNK
        if stores[b] is not None:
            stores[b].wait()
        pltpu.sync_copy(idx_hbm.at[pl.ds(off, _CHUNK)], idx_v.at[b])
        gathers[b] = pltpu.async_copy(table.at[idx_v.at[b]], rows_v.at[b], gsem[b])
        if g >= 1:
            pb = (g - 1) % 2
            poff = base + (g - 1) * _CHUNK
            gathers[pb].wait()
            stores[pb] = pltpu.async_copy(
                rows_v.at[pb], out_hbm.at[pl.ds(poff, _CHUNK)], ssem[pb])

    lb = (_N_CHUNKS - 1) % 2
    loff = base + (_N_CHUNKS - 1) * _CHUNK
    gathers[lb].wait()
    stores[lb ^ 1].wait()
    pltpu.async_copy(rows_v.at[lb], out_hbm.at[pl.ds(loff, _CHUNK)], ssem[lb]).wait()


def kernel(literal, embedding):
    flat = literal.reshape(-1)
    out = _embed_gather(flat, embedding)
    return out.reshape(literal.shape[0], literal.shape[1], D_FEAT)


# speedup vs baseline: 2.6437x; 1.7695x over previous
import functools

import jax
import jax.numpy as jnp
from jax import lax
from jax.experimental import pallas as pl
from jax.experimental.pallas import tpu as pltpu
from jax.experimental.pallas import tpu_sc as plsc

ALPHABET_SIZE = 500000
D_FEAT = 32

_info = plsc.get_sparse_core_info()
_NC, _NS, _L = _info.num_cores, _info.num_subcores, _info.num_lanes
_NW = _NC * _NS

_NB, _NT = 4096, 200

_R0 = ALPHABET_SIZE // 128
_NTILES = 3907
_ROW_BASE = _R0 * 128
_SCR_LINES = _NTILES * 32


@functools.partial(
    pl.kernel,
    mesh=plsc.VectorSubcoreMesh(core_axis_name="c", subcore_axis_name="s"),
    out_type=jax.ShapeDtypeStruct((_SCR_LINES, 128), jnp.float32),
    scratch_types=[
        pltpu.VMEM((2, 32, 128), jnp.float32),
        pltpu.VMEM((2, 32, 128), jnp.float32),
        pltpu.SemaphoreType.DMA,
        pltpu.SemaphoreType.DMA,
        pltpu.SemaphoreType.DMA,
        pltpu.SemaphoreType.DMA,
    ],
    compiler_params=pltpu.CompilerParams(
        use_tc_tiling_on_sc=True, needs_layout_passes=False),
    name="detile_table",
)
def _detile(embT, scr, bin_, bout, l0, l1, s0, s1):
    wid = lax.axis_index("s") * _NC + lax.axis_index("c")
    iota = lax.iota(jnp.int32, _L)
    lsem = [l0, l1]
    ssem = [s0, s1]

    rows_lo = iota
    rows_hi = iota + 16

    def tile_idx(j):
        return wid + _NW * j

    def start_loads(j, b):
        r = _R0 + tile_idx(j)
        for dt in range(4):
            pltpu.async_copy(
                embT.at[pl.ds(8 * dt, 8), pl.ds(128 * r, 128)],
                bin_.at[b, pl.ds(8 * dt, 8)], lsem[b])

    def drain_loads(j, b):
        r = _R0 + tile_idx(j)
        for dt in range(4):
            pltpu.make_async_copy(
                embT.at[pl.ds(8 * dt, 8), pl.ds(128 * r, 128)],
                bin_.at[b, pl.ds(8 * dt, 8)], lsem[b]).wait()

    def drain_store(j, b):
        pltpu.make_async_copy(
            bout.at[b], scr.at[pl.ds(32 * tile_idx(j), 32)], ssem[b]).wait()

    def transpose_tile(b, valid_lanes):
        @plsc.parallel_loop(0, valid_lanes // 4, unroll=4)
        def g_body(g):
            for sl in range(8):
                k = sl // 2
                rows = rows_hi if sl % 2 else rows_lo
                cols = jnp.full((_L,), 4 * g + k, jnp.int32)
                vals = plsc.load_gather(bin_.at[b], [rows, cols])
                bout[b, g, pl.ds((sl % 2) * 16 + 32 * k, 16)] = vals

    def emit(j, b):
        transpose_tile(b, 128)
        pltpu.async_copy(bout.at[b], scr.at[pl.ds(32 * tile_idx(j), 32)],
                         ssem[b])

    start_loads(0, 0)

    def step(jj, carry):
        j0 = 2 * jj
        j1 = 2 * jj + 1
        start_loads(j1, 1)
        drain_loads(j0, 0)

        @pl.when(j0 >= 2)
        def _():
            drain_store(j0 - 2, 0)
        emit(j0, 0)

        @pl.when(j0 < 120)
        def _():
            start_loads(j0 + 2, 0)
        drain_loads(j1, 1)

        @pl.when(j1 >= 3)
        def _():
            drain_store(j1 - 2, 1)
        emit(j1, 1)
        return carry

    lax.fori_loop(0, 61, step, 0)
    drain_store(120, 0)
    drain_store(121, 1)

    def tail_full(i):
        r = _R0 + i
        for dt in range(4):
            pltpu.sync_copy(
                embT.at[pl.ds(8 * dt, 8), pl.ds(128 * r, 128)],
                bin_.at[0, pl.ds(8 * dt, 8)])
        transpose_tile(0, 128)
        pltpu.sync_copy(bout.at[0], scr.at[pl.ds(32 * i, 32)])

    @pl.when(wid == 0)
    def _():
        tail_full(3904)

    @pl.when(wid == 1)
    def _():
        tail_full(3905)

    @pl.when(wid == 2)
    def _():
        for dt in range(4):
            pltpu.sync_copy(
                embT.at[pl.ds(8 * dt, 8), pl.ds(128 * (_R0 + _NTILES - 1), 64)],
                bin_.at[0, pl.ds(8 * dt, 8), pl.ds(0, 64)])
        transpose_tile(0, 64)
        pltpu.sync_copy(bout.at[0, pl.ds(0, 16)],
                        scr.at[pl.ds(32 * (_NTILES - 1), 16)])


_UNITS_PER_W = (_NT * (_NB // 128)) // _NW


@functools.partial(
    pl.kernel,
    mesh=plsc.VectorSubcoreMesh(core_axis_name="c", subcore_axis_name="s"),
    out_type=jax.ShapeDtypeStruct((_NT, 4, _NB // 128, 8, 128), jnp.float32),
    scratch_types=[
        pltpu.VMEM((_UNITS_PER_W * 128,), jnp.int32),
        pltpu.VMEM((2, 512, D_FEAT), jnp.float32),
        pltpu.VMEM((4, 32, 128), jnp.float32),
        pltpu.SemaphoreType.DMA,
        pltpu.SemaphoreType.DMA,
        pltpu.SemaphoreType.DMA,
        pltpu.SemaphoreType.DMA,
        pltpu.SemaphoreType.DMA,
        pltpu.SemaphoreType.DMA,
    ],
    compiler_params=pltpu.CompilerParams(
        use_tc_tiling_on_sc=False, needs_layout_passes=False),
    name="gather_emit",
)
def _gather_emit(scr_rows, idxf, out4, idx_all, rows_v, obuf,
                 g0, g1, w0, w1, w2, w3):
    wid = lax.axis_index("s") * _NC + lax.axis_index("c")
    base_u = wid * _UNITS_PER_W
    iota = lax.iota(jnp.int32, _L)
    table = scr_rows.at[pl.ds(ALPHABET_SIZE - _ROW_BASE, ALPHABET_SIZE)]
    gsem = [g0, g1]
    wsem = [w0, w1, w2, w3]
    _NCHUNK = _UNITS_PER_W // 4

    pltpu.sync_copy(idxf.at[pl.ds(base_u * 128, _UNITS_PER_W * 128)], idx_all)

    def idx_ref(c):
        return idx_all.at[pl.ds(512 * c, 512)]

    def start_gather(c, b):
        pltpu.async_copy(table.at[idx_ref(c)], rows_v.at[b], gsem[b])

    def drain_gather(c, b):
        pltpu.make_async_copy(table.at[idx_ref(c)], rows_v.at[b], gsem[b]).wait()

    def drain_writes(q):
        for _i in range(4):
            pltpu.make_async_copy(
                out4.at[0, 0, 0], obuf.at[q, pl.ds(0, 8)], wsem[q]).wait()

    rows_m = [iota + 16 * m for m in range(8)]

    def transpose_emit(c, b, q):
        @plsc.parallel_loop(0, 32, unroll=4)
        def d_body(d):
            cols = jnp.full((_L,), d, jnp.int32)
            for m in range(8):
                vals = plsc.load_gather(
                    rows_v.at[b], [rows_m[m] + 128 * q, cols])
                obuf[q, d, pl.ds(16 * m, 16)] = vals
        u = base_u + 4 * c + q
        t = u // 32
        bt = u % 32
        for dt in range(4):
            pltpu.async_copy(obuf.at[q, pl.ds(8 * dt, 8)],
                             out4.at[t, dt, bt], wsem[q])

    start_gather(0, 0)

    def body(jj, carry):
        c0 = 2 * jj
        c1 = 2 * jj + 1
        start_gather(c1, 1)
        drain_gather(c0, 0)
        for q in range(4):
            @pl.when(c0 > 0)
            def _(q=q):
                drain_writes(q)
            transpose_emit(c0, 0, q)

        @pl.when(jj < _NCHUNK // 2 - 1)
        def _():
            start_gather(c0 + 2, 0)
        drain_gather(c1, 1)
        for q in range(4):
            drain_writes(q)
            transpose_emit(c1, 1, q)
        return carry

    lax.fori_loop(0, _NCHUNK // 2, body, 0)
    for q in range(4):
        drain_writes(q)


def kernel(literal, embedding):
    scr = _detile(embedding.T)
    idxf = literal.T.reshape(-1)
    out4 = _gather_emit(scr.reshape(_SCR_LINES * 4, D_FEAT), idxf)
    return out4.transpose(2, 4, 0, 1, 3).reshape(_NB, _NT, D_FEAT)
